# baseline (speedup 1.0000x reference)
import functools

import jax
import jax.numpy as jnp
from jax import lax
from jax.experimental import pallas as pl
from jax.experimental.pallas import tpu as pltpu
from jax.experimental.pallas import tpu_sc as plsc

VOCAB = 1000
VMAIN = 896
VTAIL = 128
VREM = VOCAB - VMAIN
BATCH = 1024
SEQ = 50
SEQA = 48
SEQP = 56
NUM_CORES = 2
NUM_SUBCORES = 16
NUM_WORKERS = NUM_CORES * NUM_SUBCORES
B_PER_W = BATCH // NUM_WORKERS
NBUF = 2
LANES = 16

_mesh = plsc.VectorSubcoreMesh(core_axis_name="c", subcore_axis_name="s")


@functools.partial(
    pl.kernel,
    out_type=jax.ShapeDtypeStruct((BATCH, SEQ, VOCAB), jnp.float32),
    mesh=_mesh,
    compiler_params=pltpu.CompilerParams(
        use_tc_tiling_on_sc=True, needs_layout_passes=False
    ),
    scratch_types=[
        pltpu.VMEM((B_PER_W * SEQP,), jnp.int32),
        pltpu.VMEM((NBUF, SEQA, VOCAB), jnp.float32),
        pltpu.VMEM((SEQ - SEQA, VOCAB), jnp.float32),
        pltpu.VMEM((8, VMAIN), jnp.float32),
        pltpu.VMEM((SEQA, VTAIL), jnp.float32),
        pltpu.VMEM((8, VTAIL), jnp.float32),
        pltpu.SemaphoreType.DMA,
        pltpu.SemaphoreType.DMA,
        pltpu.SemaphoreType.DMA,
        pltpu.SemaphoreType.DMA,
        pltpu.SemaphoreType.DMA,
    ],
)
def _embedding_gather(
    main_hbm, tail_hbm, idx_hbm, out_hbm,
    idx_v, stag_m, stag2, stag_b, stag_t, stag_tb,
    sm0, sm1, st, sb, stb,
):
    wid = lax.axis_index("s") * NUM_CORES + lax.axis_index("c")
    base = wid * B_PER_W
    sems_m = (sm0, sm1)

    pltpu.sync_copy(idx_hbm.at[pl.ds(base * SEQP, B_PER_W * SEQP)], idx_v)

    def gather_descs(c, b):
        idx48 = idx_v.at[pl.ds(c * SEQP, SEQA)]
        idx8 = idx_v.at[pl.ds(c * SEQP + SEQA, 8)]
        return (
            pltpu.make_async_copy(
                main_hbm.at[idx48], stag_m.at[b].at[:, pl.ds(0, VMAIN)], sems_m[b]
            ),
            pltpu.make_async_copy(tail_hbm.at[idx48], stag_t, st),
            pltpu.make_async_copy(main_hbm.at[idx8], stag_b, sb),
            pltpu.make_async_copy(tail_hbm.at[idx8], stag_tb, stb),
        )

    def start_gathers(c, b):
        idx48 = idx_v.at[pl.ds(c * SEQP, SEQA)]
        idx8 = idx_v.at[pl.ds(c * SEQP + SEQA, 8)]
        pltpu.async_copy(
            main_hbm.at[idx48], stag_m.at[b].at[:, pl.ds(0, VMAIN)], sems_m[b]
        )
        pltpu.async_copy(tail_hbm.at[idx48], stag_t, st)
        pltpu.async_copy(main_hbm.at[idx8], stag_b, sb)
        pltpu.async_copy(tail_hbm.at[idx8], stag_tb, stb)

    def wait_gathers(c, b):
        for d in gather_descs(c, b):
            d.wait()

    lane = lax.iota(jnp.int32, LANES)
    rem_cols = VMAIN + (VREM // LANES) * LANES + lane
    rem_mask = rem_cols < VOCAB
    KREM = VREM // LANES

    def repack(b):
        @pl.loop(0, SEQA)
        def _(r):
            for k in range(KREM):
                stag_m.at[b][r, pl.ds(VMAIN + k * LANES, LANES)] = (
                    stag_t[r, pl.ds(k * LANES, LANES)]
                )
            x = stag_t[r, pl.ds(KREM * LANES, LANES)]
            row_ids = jnp.full((LANES,), r, jnp.int32)
            plsc.store_scatter(
                stag_m.at[b], [row_ids, rem_cols], x, mask=rem_mask
            )
        for r in range(SEQ - SEQA):
            for k in range(VMAIN // LANES):
                stag2[r, pl.ds(k * LANES, LANES)] = (
                    stag_b[r, pl.ds(k * LANES, LANES)]
                )
            for k in range(KREM):
                stag2[r, pl.ds(VMAIN + k * LANES, LANES)] = (
                    stag_tb[r, pl.ds(k * LANES, LANES)]
                )
            x = stag_tb[r, pl.ds(KREM * LANES, LANES)]
            row_ids = jnp.full((LANES,), r, jnp.int32)
            plsc.store_scatter(stag2, [row_ids, rem_cols], x, mask=rem_mask)

    def write_out(c, b):
        out_slab = out_hbm.at[base + c]
        pltpu.sync_copy(stag_m.at[b], out_slab.at[pl.ds(0, SEQA)])
        pltpu.sync_copy(stag2, out_slab.at[pl.ds(SEQA, SEQ - SEQA)])

    start_gathers(0, 0)

    @pl.loop(0, B_PER_W - 2, step=2)
    def _(g):
        for b in range(2):
            c = g + b
            wait_gathers(c, b)
            repack(b)
            start_gathers(c + 1, 1 - b)
            write_out(c, b)

    c = B_PER_W - 2
    wait_gathers(c, 0)
    repack(0)
    start_gathers(c + 1, 1)
    write_out(c, 0)
    wait_gathers(c + 1, 1)
    repack(1)
    write_out(c + 1, 1)


def kernel(index, token_embedding_table):
    table_main = token_embedding_table[:, :VMAIN]
    table_tail = jnp.pad(
        token_embedding_table[:, VMAIN:], ((0, 0), (0, VTAIL - VREM))
    )
    idxp = jnp.pad(index, ((0, 0), (0, SEQP - SEQ))).reshape(-1)
    out = _embedding_gather(table_main, table_tail, idxp)
    return out, None
